# Initial kernel scaffold; baseline (speedup 1.0000x reference)
import jax
import jax.numpy as jnp
from jax.experimental import pallas as pl


def kernel(x, edge_index, edge_attr, W1, b1, W2, b2):
    raise NotImplementedError("write your pallas kernel here")


# speedup vs baseline: 2.4686x; 2.4686x over previous
import functools

import jax
import jax.numpy as jnp
from jax import lax
from jax.experimental import pallas as pl
from jax.experimental.pallas import tpu as pltpu
from jax.experimental.pallas import tpu_sc as plsc

_NC = 2
_NS = 16
_L = 16

_EC = 80
_IC = 128


@functools.lru_cache(maxsize=None)
def _make_aggregate(N, NP, E, Dh):
    n_per_tile = NP // _NS
    e_per_tile = E // _NS
    n_chunks_init = n_per_tile // _IC
    n_chunks_edge = e_per_tile // _EC
    assert n_per_tile % _IC == 0 and e_per_tile % _EC == 0

    mesh = plsc.VectorSubcoreMesh(core_axis_name="c", subcore_axis_name="s")

    def body(x2_hbm, ea2_hbm, src_hbm, dst_hbm, out_hbm,
             haggr, buf_n, buf_x, buf_e, idxn, idxx, idxe, srcb, dstb,
             sem_a, sem_b):
        k = lax.axis_index("c")
        t = lax.axis_index("s")
        lanes = jnp.arange(_L, dtype=jnp.int32)

        def init_chunk(c, _):
            base = t * n_per_tile + c * _IC
            for i in range(_IC // _L):
                node = jnp.minimum(base + i * _L + lanes, N - 1)
                idxn[pl.ds(i * _L, _L)] = node * 2 + k
            pltpu.async_copy(x2_hbm.at[idxn], buf_n, sem_a).wait()
            pltpu.sync_copy(buf_n, haggr.at[pl.ds(base, _IC)])
            return 0

        lax.fori_loop(0, n_chunks_init, init_chunk, 0)
        plsc.subcore_barrier()

        def edge_chunk(c, _):
            e0 = t * e_per_tile + c * _EC
            pltpu.sync_copy(src_hbm.at[pl.ds(e0, _EC)], srcb)
            pltpu.sync_copy(dst_hbm.at[pl.ds(e0, _EC)], dstb)
            for i in range(_EC // _L):
                s = srcb[pl.ds(i * _L, _L)]
                idxx[pl.ds(i * _L, _L)] = s * 2 + k
                idxe[pl.ds(i * _L, _L)] = (e0 + i * _L + lanes) * 2 + k
            cx = pltpu.async_copy(x2_hbm.at[idxx], buf_x, sem_a)
            ce = pltpu.async_copy(ea2_hbm.at[idxe], buf_e, sem_b)
            cx.wait()
            ce.wait()

            def row(r, _):
                for j in range(Dh // _L):
                    sl = pl.ds(j * _L, _L)
                    buf_x[r, sl] = jnp.maximum(buf_x[r, sl] + buf_e[r, sl], 0.0)
                return 0

            lax.fori_loop(0, _EC, row, 0)
            pltpu.sync_copy(buf_x, haggr.at[dstb], add=True)
            return 0

        lax.fori_loop(0, n_chunks_edge, edge_chunk, 0)
        plsc.subcore_barrier()

        base = t * n_per_tile
        pltpu.sync_copy(haggr.at[pl.ds(base, n_per_tile)],
                        out_hbm.at[k, pl.ds(base, n_per_tile)])

    return pl.kernel(
        body,
        out_type=jax.ShapeDtypeStruct((_NC, NP, Dh), jnp.float32),
        mesh=mesh,
        scratch_types=[
            pltpu.VMEM_SHARED((NP, Dh), jnp.float32),
            pltpu.VMEM((_IC, Dh), jnp.float32),
            pltpu.VMEM((_EC, Dh), jnp.float32),
            pltpu.VMEM((_EC, Dh), jnp.float32),
            pltpu.VMEM((_IC,), jnp.int32),
            pltpu.VMEM((_EC,), jnp.int32),
            pltpu.VMEM((_EC,), jnp.int32),
            pltpu.VMEM((_EC,), jnp.int32),
            pltpu.VMEM((_EC,), jnp.int32),
            pltpu.SemaphoreType.DMA,
            pltpu.SemaphoreType.DMA,
        ],
    )


def _mlp_body(h0_ref, h1_ref, w1a_ref, w1b_ref, b1_ref, w2_ref, b2_ref, out_ref):
    h0 = h0_ref[0]
    h1 = h1_ref[0]
    tm = jnp.dot(h0, w1a_ref[...], preferred_element_type=jnp.float32)
    tm = tm + jnp.dot(h1, w1b_ref[...], preferred_element_type=jnp.float32)
    tm = jnp.maximum(tm + b1_ref[...], 0.0)
    out_ref[...] = jnp.dot(tm, w2_ref[...], preferred_element_type=jnp.float32) + b2_ref[...]


@functools.lru_cache(maxsize=None)
def _make_mlp(N, NP, D, R=1000):
    Dh = D // 2
    grid = (N // R,)
    return pl.pallas_call(
        _mlp_body,
        grid=grid,
        in_specs=[
            pl.BlockSpec((1, R, Dh), lambda i: (0, i, 0)),
            pl.BlockSpec((1, R, Dh), lambda i: (1, i, 0)),
            pl.BlockSpec((Dh, D), lambda i: (0, 0)),
            pl.BlockSpec((Dh, D), lambda i: (1, 0)),
            pl.BlockSpec((1, D), lambda i: (0, 0)),
            pl.BlockSpec((D, D), lambda i: (0, 0)),
            pl.BlockSpec((1, D), lambda i: (0, 0)),
        ],
        out_specs=pl.BlockSpec((R, D), lambda i: (i, 0)),
        out_shape=jax.ShapeDtypeStruct((N, D), jnp.float32),
    )


def kernel(x, edge_index, edge_attr, W1, b1, W2, b2):
    N, D = x.shape
    E = edge_attr.shape[0]
    Dh = D // 2
    NP = ((N + _NS * _IC - 1) // (_NS * _IC)) * (_NS * _IC)
    x2 = x.reshape(N * 2, Dh)
    ea2 = edge_attr.reshape(E * 2, Dh)
    src = edge_index[0]
    dst = edge_index[1]
    h2 = _make_aggregate(N, NP, E, Dh)(x2, ea2, src, dst)
    out = _make_mlp(N, NP, D)(h2, h2, W1, W1, b1.reshape(1, D), W2, b2.reshape(1, D))
    return out


# speedup vs baseline: 3.7849x; 1.5332x over previous
import functools

import jax
import jax.numpy as jnp
from jax import lax
from jax.experimental import pallas as pl
from jax.experimental.pallas import tpu as pltpu
from jax.experimental.pallas import tpu_sc as plsc

_NC = 2
_NS = 16
_L = 16

_EC = 80
_CB = 25
_NB = 5


@functools.lru_cache(maxsize=None)
def _make_aggregate(N, NP, E, Dh):
    n_per_tile = NP // _NS
    e_per_tile = E // _NS
    n_chunks_init = n_per_tile // _EC
    assert n_per_tile % _EC == 0
    assert e_per_tile == _NB * _CB * _EC and _CB % 2 == 1

    mesh = plsc.VectorSubcoreMesh(core_axis_name="c", subcore_axis_name="s")

    def body(x2_hbm, ea2_hbm, src_hbm, dst_hbm, out_hbm,
             haggr, bufx0, bufx1, bufe0, bufe1,
             idxx0, idxx1, idxe0, idxe1, dstb0, dstb1,
             src_b, dst_b,
             semx0, semx1, seme0, seme1):
        k = lax.axis_index("c")
        t = lax.axis_index("s")
        lanes = jnp.arange(_L, dtype=jnp.int32)

        def init_chunk(c, _):
            base = t * n_per_tile + c * _EC
            for i in range(_EC // _L):
                node = jnp.minimum(base + i * _L + lanes, N - 1)
                idxx0[pl.ds(i * _L, _L)] = node * 2 + k
            pltpu.async_copy(x2_hbm.at[idxx0], bufx0, semx0).wait()
            pltpu.sync_copy(bufx0, haggr.at[pl.ds(base, _EC)])
            return 0

        lax.fori_loop(0, n_chunks_init, init_chunk, 0)
        plsc.subcore_barrier()

        def build_idx(lc, idxx, dstb):
            for i in range(_EC // _L):
                sl = pl.ds(i * _L, _L)
                bsl = pl.ds(lc * _EC + i * _L, _L)
                idxx[sl] = src_b[bsl] * 2 + k
                dstb[sl] = dst_b[bsl]

        def issue_gather(e0, idxx, idxe, bufx, bufe, semx, seme):
            for i in range(_EC // _L):
                sl = pl.ds(i * _L, _L)
                idxe[sl] = (e0 + i * _L + lanes) * 2 + k
            pltpu.async_copy(x2_hbm.at[idxx], bufx, semx)
            pltpu.async_copy(ea2_hbm.at[idxe], bufe, seme)

        def wait_gather(idxx, idxe, bufx, bufe, semx, seme):
            pltpu.make_async_copy(x2_hbm.at[idxx], bufx, semx).wait()
            pltpu.make_async_copy(ea2_hbm.at[idxe], bufe, seme).wait()

        def compute(bufx, bufe):
            def rows(i, _):
                for rr in range(4):
                    r = i * 4 + rr
                    for j in range(Dh // _L):
                        sl = pl.ds(j * _L, _L)
                        bufx[r, sl] = jnp.maximum(bufx[r, sl] + bufe[r, sl], 0.0)
                return 0
            lax.fori_loop(0, _EC // 4, rows, 0)

        def scatter(bufx, dstb):
            pltpu.sync_copy(bufx, haggr.at[dstb], add=True)

        def batch(b, _):
            pltpu.sync_copy(src_hbm.at[t, b], src_b)
            pltpu.sync_copy(dst_hbm.at[t, b], dst_b)
            e_base = (t * _NB + b) * _CB * _EC

            build_idx(0, idxx0, dstb0)
            issue_gather(e_base, idxx0, idxe0, bufx0, bufe0, semx0, seme0)

            def pair(j, _):
                c0 = j * 2
                build_idx(c0 + 1, idxx1, dstb1)
                issue_gather(e_base + (c0 + 1) * _EC,
                             idxx1, idxe1, bufx1, bufe1, semx1, seme1)
                wait_gather(idxx0, idxe0, bufx0, bufe0, semx0, seme0)
                compute(bufx0, bufe0)
                scatter(bufx0, dstb0)
                build_idx(c0 + 2, idxx0, dstb0)
                issue_gather(e_base + (c0 + 2) * _EC,
                             idxx0, idxe0, bufx0, bufe0, semx0, seme0)
                wait_gather(idxx1, idxe1, bufx1, bufe1, semx1, seme1)
                compute(bufx1, bufe1)
                scatter(bufx1, dstb1)
                return 0

            lax.fori_loop(0, (_CB - 1) // 2, pair, 0)

            wait_gather(idxx0, idxe0, bufx0, bufe0, semx0, seme0)
            compute(bufx0, bufe0)
            scatter(bufx0, dstb0)
            return 0

        lax.fori_loop(0, _NB, batch, 0)
        plsc.subcore_barrier()

        base = t * n_per_tile
        pltpu.sync_copy(haggr.at[pl.ds(base, n_per_tile)],
                        out_hbm.at[k, pl.ds(base, n_per_tile)])

    return pl.kernel(
        body,
        out_type=jax.ShapeDtypeStruct((_NC, NP, Dh), jnp.float32),
        mesh=mesh,
        scratch_types=[
            pltpu.VMEM_SHARED((NP, Dh), jnp.float32),
            pltpu.VMEM((_EC, Dh), jnp.float32),
            pltpu.VMEM((_EC, Dh), jnp.float32),
            pltpu.VMEM((_EC, Dh), jnp.float32),
            pltpu.VMEM((_EC, Dh), jnp.float32),
            pltpu.VMEM((_EC,), jnp.int32),
            pltpu.VMEM((_EC,), jnp.int32),
            pltpu.VMEM((_EC,), jnp.int32),
            pltpu.VMEM((_EC,), jnp.int32),
            pltpu.VMEM((_EC,), jnp.int32),
            pltpu.VMEM((_EC,), jnp.int32),
            pltpu.VMEM((_CB * _EC,), jnp.int32),
            pltpu.VMEM((_CB * _EC,), jnp.int32),
            pltpu.SemaphoreType.DMA,
            pltpu.SemaphoreType.DMA,
            pltpu.SemaphoreType.DMA,
            pltpu.SemaphoreType.DMA,
        ],
    )


def _mlp_body(h0_ref, h1_ref, w1a_ref, w1b_ref, b1_ref, w2_ref, b2_ref, out_ref):
    h0 = h0_ref[0]
    h1 = h1_ref[0]
    tm = jnp.dot(h0, w1a_ref[...], preferred_element_type=jnp.float32)
    tm = tm + jnp.dot(h1, w1b_ref[...], preferred_element_type=jnp.float32)
    tm = jnp.maximum(tm + b1_ref[...], 0.0)
    out_ref[...] = jnp.dot(tm, w2_ref[...], preferred_element_type=jnp.float32) + b2_ref[...]


@functools.lru_cache(maxsize=None)
def _make_mlp(N, NP, D, R=1000):
    Dh = D // 2
    grid = (N // R,)
    return pl.pallas_call(
        _mlp_body,
        grid=grid,
        in_specs=[
            pl.BlockSpec((1, R, Dh), lambda i: (0, i, 0)),
            pl.BlockSpec((1, R, Dh), lambda i: (1, i, 0)),
            pl.BlockSpec((Dh, D), lambda i: (0, 0)),
            pl.BlockSpec((Dh, D), lambda i: (1, 0)),
            pl.BlockSpec((1, D), lambda i: (0, 0)),
            pl.BlockSpec((D, D), lambda i: (0, 0)),
            pl.BlockSpec((1, D), lambda i: (0, 0)),
        ],
        out_specs=pl.BlockSpec((R, D), lambda i: (i, 0)),
        out_shape=jax.ShapeDtypeStruct((N, D), jnp.float32),
    )


def kernel(x, edge_index, edge_attr, W1, b1, W2, b2):
    N, D = x.shape
    E = edge_attr.shape[0]
    Dh = D // 2
    NP = ((N + _NS * _EC - 1) // (_NS * _EC)) * (_NS * _EC)
    x2 = x.reshape(N * 2, Dh)
    ea2 = edge_attr.reshape(E * 2, Dh)
    src = edge_index[0].reshape(_NS, _NB, _CB * _EC)
    dst = edge_index[1].reshape(_NS, _NB, _CB * _EC)
    h2 = _make_aggregate(N, NP, E, Dh)(x2, ea2, src, dst)
    out = _make_mlp(N, NP, D)(h2, h2, W1, W1, b1.reshape(1, D), W2, b2.reshape(1, D))
    return out


# speedup vs baseline: 3.8616x; 1.0203x over previous
import functools

import jax
import jax.numpy as jnp
from jax import lax
from jax.experimental import pallas as pl
from jax.experimental.pallas import tpu as pltpu
from jax.experimental.pallas import tpu_sc as plsc

_NC = 2
_NS = 16
_L = 16

_EC = 80
_CB = 25
_NB = 5


@functools.lru_cache(maxsize=None)
def _make_aggregate(N, NP, E, Dh):
    n_per_tile = NP // _NS
    e_per_tile = E // _NS
    n_chunks_init = n_per_tile // _EC
    assert n_per_tile % _EC == 0
    assert e_per_tile == _NB * _CB * _EC and _CB % 2 == 1

    mesh = plsc.VectorSubcoreMesh(core_axis_name="c", subcore_axis_name="s")

    def body(x2_hbm, ea2_hbm, sd_hbm, out_hbm,
             haggr, bufx0, bufx1, bufe0, bufe1,
             idxx0, idxx1, idxe0, idxe1, dstb0, dstb1,
             sd_b,
             semx0, semx1, seme0, seme1):
        k = lax.axis_index("c")
        t = lax.axis_index("s")
        lanes = jnp.arange(_L, dtype=jnp.int32)

        def init_chunk(c, _):
            base = t * n_per_tile + c * _EC
            for i in range(_EC // _L):
                node = jnp.minimum(base + i * _L + lanes, N - 1)
                idxx0[pl.ds(i * _L, _L)] = node * 2 + k
            pltpu.async_copy(x2_hbm.at[idxx0], bufx0, semx0).wait()
            pltpu.sync_copy(bufx0, haggr.at[pl.ds(base, _EC)])
            return 0

        lax.fori_loop(0, n_chunks_init, init_chunk, 0)
        plsc.subcore_barrier()

        def build_idx(lc, idxx, dstb):
            for i in range(_EC // _L):
                sl = pl.ds(i * _L, _L)
                idxx[sl] = sd_b[0, pl.ds(lc * _EC + i * _L, _L)] * 2 + k
                dstb[sl] = sd_b[1, pl.ds(lc * _EC + i * _L, _L)]

        def issue_gather(e0, idxx, idxe, bufx, bufe, semx, seme):
            for i in range(_EC // _L):
                idxe[pl.ds(i * _L, _L)] = (e0 + i * _L + lanes) * 2 + k
            pltpu.async_copy(x2_hbm.at[idxx], bufx, semx)
            pltpu.async_copy(ea2_hbm.at[idxe], bufe, seme)

        def wait_gather(e0, idxx, idxe, bufx, bufe, semx, seme):
            pltpu.make_async_copy(x2_hbm.at[idxx], bufx, semx).wait()
            pltpu.make_async_copy(ea2_hbm.at[idxe], bufe, seme).wait()

        def compute(bufx, bufe):
            def rows(i, _):
                for rr in range(4):
                    r = i * 4 + rr
                    for j in range(Dh // _L):
                        sl = pl.ds(j * _L, _L)
                        bufx[r, sl] = jnp.maximum(bufx[r, sl] + bufe[r, sl], 0.0)
                return 0
            lax.fori_loop(0, _EC // 4, rows, 0)

        def scatter(bufx, dstb):
            pltpu.sync_copy(bufx, haggr.at[dstb], add=True)

        def batch(b, _):
            pltpu.sync_copy(sd_hbm.at[t, b], sd_b)
            e_base = (t * _NB + b) * _CB * _EC

            build_idx(0, idxx0, dstb0)
            issue_gather(e_base, idxx0, idxe0, bufx0, bufe0, semx0, seme0)

            def pair(j, _):
                c0 = j * 2
                e1 = e_base + (c0 + 1) * _EC
                e2 = e_base + (c0 + 2) * _EC
                build_idx(c0 + 1, idxx1, dstb1)
                issue_gather(e1, idxx1, idxe1, bufx1, bufe1, semx1, seme1)
                wait_gather(e1 - _EC, idxx0, idxe0, bufx0, bufe0, semx0, seme0)
                compute(bufx0, bufe0)
                scatter(bufx0, dstb0)
                build_idx(c0 + 2, idxx0, dstb0)
                issue_gather(e2, idxx0, idxe0, bufx0, bufe0, semx0, seme0)
                wait_gather(e1, idxx1, idxe1, bufx1, bufe1, semx1, seme1)
                compute(bufx1, bufe1)
                scatter(bufx1, dstb1)
                return 0

            lax.fori_loop(0, (_CB - 1) // 2, pair, 0)

            wait_gather(e_base + (_CB - 1) * _EC, idxx0, idxe0, bufx0, bufe0, semx0, seme0)
            compute(bufx0, bufe0)
            scatter(bufx0, dstb0)
            return 0

        lax.fori_loop(0, _NB, batch, 0)
        plsc.subcore_barrier()

        base = t * n_per_tile
        pltpu.sync_copy(haggr.at[pl.ds(base, n_per_tile)],
                        out_hbm.at[k, pl.ds(base, n_per_tile)])

    return pl.kernel(
        body,
        out_type=jax.ShapeDtypeStruct((_NC, NP, Dh), jnp.float32),
        mesh=mesh,
        scratch_types=[
            pltpu.VMEM_SHARED((NP, Dh), jnp.float32),
            pltpu.VMEM((_EC, Dh), jnp.float32),
            pltpu.VMEM((_EC, Dh), jnp.float32),
            pltpu.VMEM((_EC, Dh), jnp.float32),
            pltpu.VMEM((_EC, Dh), jnp.float32),
            pltpu.VMEM((_EC,), jnp.int32),
            pltpu.VMEM((_EC,), jnp.int32),
            pltpu.VMEM((_EC,), jnp.int32),
            pltpu.VMEM((_EC,), jnp.int32),
            pltpu.VMEM((_EC,), jnp.int32),
            pltpu.VMEM((_EC,), jnp.int32),
            pltpu.VMEM((2, _CB * _EC), jnp.int32),
            pltpu.SemaphoreType.DMA,
            pltpu.SemaphoreType.DMA,
            pltpu.SemaphoreType.DMA,
            pltpu.SemaphoreType.DMA,
        ],
    )


def _mlp_body(h0_ref, h1_ref, w1a_ref, w1b_ref, b1_ref, w2_ref, b2_ref, out_ref):
    h0 = h0_ref[0]
    h1 = h1_ref[0]
    tm = jnp.dot(h0, w1a_ref[...], preferred_element_type=jnp.float32)
    tm = tm + jnp.dot(h1, w1b_ref[...], preferred_element_type=jnp.float32)
    tm = jnp.maximum(tm + b1_ref[...], 0.0)
    out_ref[...] = jnp.dot(tm, w2_ref[...], preferred_element_type=jnp.float32) + b2_ref[...]


@functools.lru_cache(maxsize=None)
def _make_mlp(N, NP, D, R=1000):
    Dh = D // 2
    grid = (N // R,)
    return pl.pallas_call(
        _mlp_body,
        grid=grid,
        in_specs=[
            pl.BlockSpec((1, R, Dh), lambda i: (0, i, 0)),
            pl.BlockSpec((1, R, Dh), lambda i: (1, i, 0)),
            pl.BlockSpec((Dh, D), lambda i: (0, 0)),
            pl.BlockSpec((Dh, D), lambda i: (1, 0)),
            pl.BlockSpec((1, D), lambda i: (0, 0)),
            pl.BlockSpec((D, D), lambda i: (0, 0)),
            pl.BlockSpec((1, D), lambda i: (0, 0)),
        ],
        out_specs=pl.BlockSpec((R, D), lambda i: (i, 0)),
        out_shape=jax.ShapeDtypeStruct((N, D), jnp.float32),
    )


def kernel(x, edge_index, edge_attr, W1, b1, W2, b2):
    N, D = x.shape
    E = edge_attr.shape[0]
    Dh = D // 2
    NP = ((N + _NS * _EC - 1) // (_NS * _EC)) * (_NS * _EC)
    x2 = x.reshape(N * 2, Dh)
    ea2 = edge_attr.reshape(E * 2, Dh)
    sd = edge_index.reshape(2, _NS, _NB, _CB * _EC).transpose(1, 2, 0, 3)
    h2 = _make_aggregate(N, NP, E, Dh)(x2, ea2, sd)
    out = _make_mlp(N, NP, D)(h2, h2, W1, W1, b1.reshape(1, D), W2, b2.reshape(1, D))
    return out
